# baseline (speedup 1.0000x reference)
import functools

import jax
import jax.numpy as jnp
from jax import lax
from jax.experimental import pallas as pl
from jax.experimental.pallas import tpu as pltpu
from jax.experimental.pallas import tpu_sc as plsc

VOCAB = 1000000
MAXLEN = 200
EMBED = 64
BATCH = 1024

NC, NS = 2, 16
NW = NC * NS
SPW = BATCH // NW
B = BATCH * MAXLEN
G1 = 128
G2 = MAXLEN - G1
VPR = EMBED // 16
NB = 3


@functools.partial(
    pl.kernel,
    mesh=plsc.VectorSubcoreMesh(core_axis_name="c", subcore_axis_name="s"),
    out_type=jax.ShapeDtypeStruct((B, EMBED), jnp.float32),
    scratch_types=[
        pltpu.VMEM((NB, MAXLEN), jnp.int32),
        pltpu.VMEM((NB, MAXLEN), jnp.int32),
        pltpu.VMEM((NB, MAXLEN, 2 * EMBED), jnp.float32),
        pltpu.VMEM((MAXLEN, EMBED), jnp.float32),
        pltpu.SemaphoreType.DMA,
        pltpu.SemaphoreType.DMA,
        pltpu.SemaphoreType.DMA,
        pltpu.SemaphoreType.DMA,
        pltpu.SemaphoreType.DMA,
        pltpu.SemaphoreType.DMA,
    ],
    compiler_params=pltpu.CompilerParams(use_tc_tiling_on_sc=False),
)
def _embed_sc(x_hbm, h_hbm, tok_hbm, pos_hbm, out_hbm,
              idx_v, idx_h, bufs, pos_v, sg0, sg1, sg2, so0, so1, so2):
    wid = lax.axis_index("s") * NC + lax.axis_index("c")
    b0 = wid * SPW
    gsems = (sg0, sg1, sg2)
    osems = (so0, so1, so2)

    pltpu.sync_copy(pos_hbm, pos_v)

    def out_drain(osem):
        pltpu.make_async_copy(
            out_hbm.at[pl.ds(0, MAXLEN), :],
            bufs.at[0, pl.ds(0, MAXLEN), pl.ds(0, EMBED)], osem).wait()

    def gather_drain(gsem):
        pltpu.make_async_copy(
            tok_hbm.at[pl.ds(0, MAXLEN), :],
            bufs.at[0, pl.ds(0, MAXLEN), :], gsem).wait()

    def fire(p, gsem):
        pltpu.async_copy(tok_hbm.at[idx_v.at[p, pl.ds(0, G1)]],
                         bufs.at[p, pl.ds(0, G1), :], gsem)
        pltpu.async_copy(tok_hbm.at[idx_v.at[p, pl.ds(G1, G2)]],
                         bufs.at[p, pl.ds(G1, G2), :], gsem)

    def step(s, carry):
        p = lax.rem(s, NB)
        f = lax.rem(s + 1, NB)
        live = s < SPW
        fin = jnp.logical_and(s >= 2, True)

        for i in range(NB):
            @pl.when(jnp.logical_and(s >= NB, p == i))
            def _(i=i):
                out_drain(osems[i])

        @pl.when(live)
        def _():
            b = b0 + s
            pltpu.sync_copy(x_hbm.at[pl.ds(b * MAXLEN, MAXLEN)],
                            idx_v.at[p])
            pltpu.sync_copy(h_hbm.at[pl.ds(b * MAXLEN, MAXLEN)],
                            idx_h.at[p])

        for i in range(NB):
            @pl.when(jnp.logical_and(live, p == i))
            def _(i=i):
                fire(p, gsems[i])

        for i in range(NB):
            @pl.when(jnp.logical_and(fin, f == i))
            def _(i=i):
                gather_drain(gsems[i])

        @pl.when(fin)
        def _():
            for g in range(MAXLEN // 16):
                hv = idx_h[f, pl.ds(g * 16, 16)]
                for j in range(16):
                    r = g * 16 + j
                    hb = hv[j] << 6
                    for c in range(VPR):
                        sl = pl.ds(c * 16, 16)
                        bufs.at[f, r, sl].set(
                            bufs[f, r, pl.ds(hb + c * 16, 16)]
                            + pos_v[r, sl])

        for i in range(NB):
            @pl.when(jnp.logical_and(fin, f == i))
            def _(i=i):
                pltpu.async_copy(
                    bufs.at[f, pl.ds(0, MAXLEN), pl.ds(0, EMBED)],
                    out_hbm.at[pl.ds((b0 + s - 2) * MAXLEN, MAXLEN), :],
                    osems[i])

        return carry

    lax.fori_loop(0, SPW + 2, step, 0)
    out_drain(osems[(SPW - 1) % NB])


def kernel(x, token_table, pos_table):
    xf = x.reshape(B).astype(jnp.int32)
    rows = xf >> 1
    half = xf & 1
    tok2 = token_table.reshape(VOCAB // 2, 2 * EMBED)
    out = _embed_sc(rows, half, tok2, pos_table)
    return out.reshape(BATCH, MAXLEN, EMBED)


# speedup vs baseline: 1.1600x; 1.1600x over previous
import functools

import jax
import jax.numpy as jnp
from jax import lax
from jax.experimental import pallas as pl
from jax.experimental.pallas import tpu as pltpu
from jax.experimental.pallas import tpu_sc as plsc

VOCAB = 1000000
MAXLEN = 200
EMBED = 64
BATCH = 1024

NC, NS = 2, 16
NW = NC * NS
SPW = BATCH // NW
B = BATCH * MAXLEN
G1 = 128
G2 = MAXLEN - G1
VPR = EMBED // 16
NB = 3


@functools.partial(
    pl.kernel,
    mesh=plsc.VectorSubcoreMesh(core_axis_name="c", subcore_axis_name="s"),
    out_type=jax.ShapeDtypeStruct((B, EMBED), jnp.float32),
    scratch_types=[
        pltpu.VMEM((NB, MAXLEN), jnp.int32),
        pltpu.VMEM((NB, MAXLEN, EMBED), jnp.float32),
        pltpu.VMEM((MAXLEN, EMBED), jnp.float32),
        pltpu.SemaphoreType.DMA,
        pltpu.SemaphoreType.DMA,
        pltpu.SemaphoreType.DMA,
        pltpu.SemaphoreType.DMA,
        pltpu.SemaphoreType.DMA,
        pltpu.SemaphoreType.DMA,
    ],
    compiler_params=pltpu.CompilerParams(use_tc_tiling_on_sc=False),
)
def _embed_sc(x_hbm, tok_hbm, pos_hbm, out_hbm,
              idx_v, bufs, pos_v, sg0, sg1, sg2, so0, so1, so2):
    wid = lax.axis_index("s") * NC + lax.axis_index("c")
    b0 = wid * SPW
    gsems = (sg0, sg1, sg2)
    osems = (so0, so1, so2)

    pltpu.sync_copy(pos_hbm, pos_v)

    def out_drain(osem):
        pltpu.make_async_copy(
            out_hbm.at[pl.ds(0, MAXLEN), :],
            bufs.at[0, pl.ds(0, MAXLEN), :], osem).wait()

    def gather_drain(gsem):
        pltpu.make_async_copy(
            tok_hbm.at[pl.ds(0, MAXLEN), :],
            bufs.at[0, pl.ds(0, MAXLEN), :], gsem).wait()

    def fire(p, gsem):
        pltpu.async_copy(tok_hbm.at[idx_v.at[p, pl.ds(0, G1)]],
                         bufs.at[p, pl.ds(0, G1), :], gsem)
        pltpu.async_copy(tok_hbm.at[idx_v.at[p, pl.ds(G1, G2)]],
                         bufs.at[p, pl.ds(G1, G2), :], gsem)

    def step(s, carry):
        p = lax.rem(s, NB)
        f = lax.rem(s + 1, NB)
        live = s < SPW
        fin = jnp.logical_and(s >= 2, True)

        for i in range(NB):
            @pl.when(jnp.logical_and(s >= NB, p == i))
            def _(i=i):
                out_drain(osems[i])

        @pl.when(live)
        def _():
            b = b0 + s
            pltpu.sync_copy(x_hbm.at[pl.ds(b * MAXLEN, MAXLEN)],
                            idx_v.at[p])

        for i in range(NB):
            @pl.when(jnp.logical_and(live, p == i))
            def _(i=i):
                fire(p, gsems[i])

        for i in range(NB):
            @pl.when(jnp.logical_and(fin, f == i))
            def _(i=i):
                gather_drain(gsems[i])

        @pl.when(fin)
        def _():
            for r in range(MAXLEN):
                for c in range(VPR):
                    sl = pl.ds(c * 16, 16)
                    plsc.addupdate(bufs.at[f, r, sl], pos_v[r, sl])

        for i in range(NB):
            @pl.when(jnp.logical_and(fin, f == i))
            def _(i=i):
                pltpu.async_copy(
                    bufs.at[f, pl.ds(0, MAXLEN), :],
                    out_hbm.at[pl.ds((b0 + s - 2) * MAXLEN, MAXLEN), :],
                    osems[i])

        return carry

    lax.fori_loop(0, SPW + 2, step, 0)
    out_drain(osems[(SPW - 1) % NB])


def kernel(x, token_table, pos_table):
    xf = x.reshape(B).astype(jnp.int32)
    out = _embed_sc(xf, token_table, pos_table)
    return out.reshape(BATCH, MAXLEN, EMBED)
